# baseline (speedup 1.0000x reference)
import functools

import jax
import jax.numpy as jnp
from jax import lax
from jax.experimental import pallas as pl
from jax.experimental.pallas import tpu as pltpu
from jax.experimental.pallas import tpu_sc as plsc

_B = 4096
_D = 64
_K = 8192
_B2 = 2 * _B
_COMMIT = 0.25

_BB = 1024
_KB = 512

_NC = 2
_NS = 16
_NW = _NC * _NS
_BPW = _B2 // _NW
_CH = 128
_NCH = _BPW // _CH
_ZW = _K // _NS


_W = 128


def _sweep_body(x2_ref, xsq_ref, cbsq_ref, cb_ref, bv_ref, bg_ref):
    k = pl.program_id(1)

    mm2 = lax.dot_general(
        x2_ref[...], cb_ref[...], (((1,), (1,)), ((), ())),
        preferred_element_type=jnp.float32)
    dist = (xsq_ref[...] + cbsq_ref[...]) + mm2
    d0 = dist[:, 0 * _W:1 * _W]
    d1 = dist[:, 1 * _W:2 * _W]
    d2 = dist[:, 2 * _W:3 * _W]
    d3 = dist[:, 3 * _W:4 * _W]
    m01 = jnp.minimum(d0, d1)
    i01 = jnp.where(d1 < d0, 1 * _W, 0 * _W)
    m23 = jnp.minimum(d2, d3)
    i23 = jnp.where(d3 < d2, 3 * _W, 2 * _W)
    m = jnp.minimum(m01, m23)
    i = jnp.where(m23 < m01, i23, i01)
    lane = lax.broadcasted_iota(jnp.int32, (_BB, _W), 1)
    g = (i + k * _KB) + lane
    better = (m < bv_ref[...]) | (k == 0)
    bv_ref[...] = jnp.where(better, m, bv_ref[...])
    bg_ref[...] = jnp.where(better, g, bg_ref[...])


def _sweep_pallas(x2, xsq, cbsq2d, qcb):
    return pl.pallas_call(
        _sweep_body,
        grid=(_B2 // _BB, _K // _KB),
        in_specs=[
            pl.BlockSpec((_BB, _D), lambda b, k: (b, 0)),
            pl.BlockSpec((_BB, 1), lambda b, k: (b, 0)),
            pl.BlockSpec((1, _KB), lambda b, k: (0, k)),
            pl.BlockSpec((_KB, _D), lambda b, k: (k, 0)),
        ],
        out_specs=[pl.BlockSpec((_BB, _W), lambda b, k: (b, 0)),
                   pl.BlockSpec((_BB, _W), lambda b, k: (b, 0))],
        out_shape=[jax.ShapeDtypeStruct((_B2, _W), jnp.float32),
                   jax.ShapeDtypeStruct((_B2, _W), jnp.int32)],
    )(x2, xsq, cbsq2d, qcb)


def _finish_body(bv_ref, bg_ref, idx_ref):
    bv = bv_ref[...]
    rowmin = jnp.min(bv, axis=1, keepdims=True)
    cand = jnp.where(bv == rowmin, bg_ref[...], jnp.int32(2**31 - 1))
    idx_ref[...] = jnp.min(cand, axis=1, keepdims=True)


def _finish_pallas(bv, bg):
    return pl.pallas_call(
        _finish_body,
        grid=(_B2 // _BB,),
        in_specs=[
            pl.BlockSpec((_BB, _W), lambda b: (b, 0)),
            pl.BlockSpec((_BB, _W), lambda b: (b, 0)),
        ],
        out_specs=pl.BlockSpec((_BB, 1), lambda b: (b, 0)),
        out_shape=jax.ShapeDtypeStruct((_B2, 1), jnp.int32),
    )(bv, bg)


def _argmin_pallas(x2, xsq, cbsq2d, qcb):
    bv, bk = _sweep_pallas(x2, xsq, cbsq2d, qcb)
    return _finish_pallas(bv, bk)


def _sc_body(idx_hbm, cb_hbm, out_hbm, hist_hbm,
             idx_v, rows_v, ones_v, zeros_v, hist_sh, sem):
    c = lax.axis_index("c")
    s = lax.axis_index("s")
    wid = c * _NS + s

    pltpu.sync_copy(idx_hbm.at[pl.ds(wid * _NCH, _NCH)], idx_v)

    cps = [
        pltpu.async_copy(cb_hbm.at[idx_v.at[j]],
                         rows_v.at[pl.ds(j * _CH, _CH)], sem)
        for j in range(_NCH)
    ]

    for i in range(_CH // 16):
        ones_v[pl.ds(i * 16, 16)] = jnp.ones((16,), jnp.float32)
    for i in range(_ZW // 16):
        zeros_v[pl.ds(i * 16, 16)] = jnp.zeros((16,), jnp.float32)
    pltpu.sync_copy(zeros_v, hist_sh.at[pl.ds(s * _ZW, _ZW)])
    plsc.subcore_barrier()
    for j in range(_NCH):
        pltpu.sync_copy(ones_v, hist_sh.at[idx_v.at[j]], add=True)
    plsc.subcore_barrier()

    for cp in cps:
        cp.wait()
    pltpu.sync_copy(rows_v, out_hbm.at[pl.ds(wid * _BPW, _BPW)])

    @pl.when(s == 0)
    def _write_hist():
        pltpu.sync_copy(hist_sh, hist_hbm.at[c])


def _sc_gather_hist(idx2d, qcb):
    mesh = plsc.VectorSubcoreMesh(core_axis_name="c", subcore_axis_name="s")
    run = pl.kernel(
        _sc_body,
        out_type=(
            jax.ShapeDtypeStruct((_B2, _D), jnp.float32),
            jax.ShapeDtypeStruct((_NC, _K), jnp.float32),
        ),
        mesh=mesh,
        scratch_types=[
            pltpu.VMEM((_NCH, _CH), jnp.int32),
            pltpu.VMEM((_BPW, _D), jnp.float32),
            pltpu.VMEM((_CH,), jnp.float32),
            pltpu.VMEM((_ZW,), jnp.float32),
            pltpu.VMEM_SHARED((_K,), jnp.float32),
            pltpu.SemaphoreType.DMA,
        ],
        compiler_params=pltpu.CompilerParams(use_tc_tiling_on_sc=False),
    )
    return run(idx2d, qcb)


def _loss_body(scr_ref, rib_ref, qzs_ref, qzr_ref, hist_ref, out_ref):
    scr = scr_ref[...]
    rib = rib_ref[...]
    qzs = qzs_ref[...]
    qzr = qzr_ref[...]
    n = float(_B * _D)
    m1 = jnp.sum((scr - qzs) ** 2) / n
    m2 = jnp.sum((rib - qzr) ** 2) / n
    m3 = jnp.sum((qzr - scr) ** 2) / n
    m4 = jnp.sum((qzs - rib) ** 2) / n
    scr_loss = 2.0 * _COMMIT * m1
    forward = m2 + m1 + 0.5 * m3 + 0.5 * m4
    rib_loss = 2.0 * _COMMIT * m2 + _COMMIT * forward
    p0 = hist_ref[0:1, :] * (1.0 / _B)
    p1 = hist_ref[1:2, :] * (1.0 / _B)
    e0 = -jnp.sum(p0 * jnp.log(p0 + 1e-10))
    e1 = -jnp.sum(p1 * jnp.log(p1 + 1e-10))
    out_ref[0] = scr_loss
    out_ref[1] = rib_loss
    out_ref[2] = jnp.exp(e0)
    out_ref[3] = jnp.exp(e1)


def _loss_pallas(scr, rib, qzs, qzr, hist):
    return pl.pallas_call(
        _loss_body,
        out_shape=jax.ShapeDtypeStruct((4,), jnp.float32),
        out_specs=pl.BlockSpec(memory_space=pltpu.SMEM),
    )(scr, rib, qzs, qzr, hist)


def kernel(scRNA_semantic, ribo_semantic, flag, embedding, proj_w, proj_b):
    quant_codebook = embedding @ proj_w.T + proj_b
    cb_sq = jnp.sum(quant_codebook ** 2, axis=1)
    x = jnp.concatenate([scRNA_semantic, ribo_semantic], axis=0)
    xsq = jnp.sum(x ** 2, axis=1, keepdims=True)

    idx = _argmin_pallas(-2.0 * x, xsq, cb_sq.reshape(1, _K), quant_codebook)
    idx2d = idx.reshape(_B2 // _CH, _CH)

    qz, hist = _sc_gather_hist(idx2d, quant_codebook)
    qz_s = qz[:_B]
    qz_r = qz[_B:]

    out = _loss_pallas(scRNA_semantic, ribo_semantic, qz_s, qz_r, hist)

    scRNA_loss = out[0]
    ribo_loss = out[1]
    scRNA_perplexity = out[2]
    ribo_perplexity = out[3]
    return (qz_s, qz_r, scRNA_loss, ribo_loss,
            scRNA_perplexity, ribo_perplexity)


# speedup vs baseline: 1.3258x; 1.3258x over previous
import functools

import jax
import jax.numpy as jnp
from jax import lax
from jax.experimental import pallas as pl
from jax.experimental.pallas import tpu as pltpu
from jax.experimental.pallas import tpu_sc as plsc

_B = 4096
_D = 64
_K = 8192
_B2 = 2 * _B
_COMMIT = 0.25

_BB = 4096
_KB = 512

_NC = 2
_NS = 16
_NW = _NC * _NS
_BPW = _B2 // _NW
_CH = 128
_NCH = _BPW // _CH
_ZW = _K // _NS


_W = 128


def _sweep_body(x2_ref, xsq_ref, cbsq_ref, cb_ref, bv_ref, bg_ref):
    k = pl.program_id(1)

    mm2 = lax.dot_general(
        x2_ref[...], cb_ref[...], (((1,), (1,)), ((), ())),
        preferred_element_type=jnp.float32)
    dist = (xsq_ref[...] + cbsq_ref[...]) + mm2
    d0 = dist[:, 0 * _W:1 * _W]
    d1 = dist[:, 1 * _W:2 * _W]
    d2 = dist[:, 2 * _W:3 * _W]
    d3 = dist[:, 3 * _W:4 * _W]
    m01 = jnp.minimum(d0, d1)
    i01 = jnp.where(d1 < d0, 1 * _W, 0 * _W)
    m23 = jnp.minimum(d2, d3)
    i23 = jnp.where(d3 < d2, 3 * _W, 2 * _W)
    m = jnp.minimum(m01, m23)
    i = jnp.where(m23 < m01, i23, i01)
    lane = lax.broadcasted_iota(jnp.int32, (_BB, _W), 1)
    g = (i + k * _KB) + lane
    better = (m < bv_ref[...]) | (k == 0)
    bv_ref[...] = jnp.where(better, m, bv_ref[...])
    bg_ref[...] = jnp.where(better, g, bg_ref[...])


def _sweep_pallas(x2, xsq, cbsq2d, qcb):
    return pl.pallas_call(
        _sweep_body,
        grid=(_B2 // _BB, _K // _KB),
        in_specs=[
            pl.BlockSpec((_BB, _D), lambda b, k: (b, 0)),
            pl.BlockSpec((_BB, 1), lambda b, k: (b, 0)),
            pl.BlockSpec((1, _KB), lambda b, k: (0, k)),
            pl.BlockSpec((_KB, _D), lambda b, k: (k, 0)),
        ],
        out_specs=[pl.BlockSpec((_BB, _W), lambda b, k: (b, 0)),
                   pl.BlockSpec((_BB, _W), lambda b, k: (b, 0))],
        out_shape=[jax.ShapeDtypeStruct((_B2, _W), jnp.float32),
                   jax.ShapeDtypeStruct((_B2, _W), jnp.int32)],
    )(x2, xsq, cbsq2d, qcb)


def _finish_body(bv_ref, bg_ref, idx_ref):
    bv = bv_ref[...]
    rowmin = jnp.min(bv, axis=1, keepdims=True)
    cand = jnp.where(bv == rowmin, bg_ref[...], jnp.int32(2**31 - 1))
    idx_ref[...] = jnp.min(cand, axis=1, keepdims=True)


def _finish_pallas(bv, bg):
    return pl.pallas_call(
        _finish_body,
        grid=(_B2 // _BB,),
        in_specs=[
            pl.BlockSpec((_BB, _W), lambda b: (b, 0)),
            pl.BlockSpec((_BB, _W), lambda b: (b, 0)),
        ],
        out_specs=pl.BlockSpec((_BB, 1), lambda b: (b, 0)),
        out_shape=jax.ShapeDtypeStruct((_B2, 1), jnp.int32),
    )(bv, bg)


def _argmin_pallas(x2, xsq, cbsq2d, qcb):
    bv, bk = _sweep_pallas(x2, xsq, cbsq2d, qcb)
    return _finish_pallas(bv, bk)


def _sc_body(idx_hbm, cb_hbm, out_hbm, hist_hbm,
             idx_v, rows_v, ones_v, zeros_v, hist_sh, sem):
    c = lax.axis_index("c")
    s = lax.axis_index("s")
    wid = c * _NS + s

    pltpu.sync_copy(idx_hbm.at[pl.ds(wid * _NCH, _NCH)], idx_v)

    cps = [
        pltpu.async_copy(cb_hbm.at[idx_v.at[j]],
                         rows_v.at[pl.ds(j * _CH, _CH)], sem)
        for j in range(_NCH)
    ]

    for i in range(_CH // 16):
        ones_v[pl.ds(i * 16, 16)] = jnp.ones((16,), jnp.float32)
    for i in range(_ZW // 16):
        zeros_v[pl.ds(i * 16, 16)] = jnp.zeros((16,), jnp.float32)
    pltpu.sync_copy(zeros_v, hist_sh.at[pl.ds(s * _ZW, _ZW)])
    plsc.subcore_barrier()
    for j in range(_NCH):
        pltpu.sync_copy(ones_v, hist_sh.at[idx_v.at[j]], add=True)
    plsc.subcore_barrier()

    for cp in cps:
        cp.wait()
    pltpu.sync_copy(rows_v, out_hbm.at[pl.ds(wid * _BPW, _BPW)])

    @pl.when(s == 0)
    def _write_hist():
        pltpu.sync_copy(hist_sh, hist_hbm.at[c])


def _sc_gather_hist(idx2d, qcb):
    mesh = plsc.VectorSubcoreMesh(core_axis_name="c", subcore_axis_name="s")
    run = pl.kernel(
        _sc_body,
        out_type=(
            jax.ShapeDtypeStruct((_B2, _D), jnp.float32),
            jax.ShapeDtypeStruct((_NC, _K), jnp.float32),
        ),
        mesh=mesh,
        scratch_types=[
            pltpu.VMEM((_NCH, _CH), jnp.int32),
            pltpu.VMEM((_BPW, _D), jnp.float32),
            pltpu.VMEM((_CH,), jnp.float32),
            pltpu.VMEM((_ZW,), jnp.float32),
            pltpu.VMEM_SHARED((_K,), jnp.float32),
            pltpu.SemaphoreType.DMA,
        ],
        compiler_params=pltpu.CompilerParams(use_tc_tiling_on_sc=False),
    )
    return run(idx2d, qcb)


def _loss_body(scr_ref, rib_ref, qzs_ref, qzr_ref, hist_ref, out_ref):
    scr = scr_ref[...]
    rib = rib_ref[...]
    qzs = qzs_ref[...]
    qzr = qzr_ref[...]
    n = float(_B * _D)
    m1 = jnp.sum((scr - qzs) ** 2) / n
    m2 = jnp.sum((rib - qzr) ** 2) / n
    m3 = jnp.sum((qzr - scr) ** 2) / n
    m4 = jnp.sum((qzs - rib) ** 2) / n
    scr_loss = 2.0 * _COMMIT * m1
    forward = m2 + m1 + 0.5 * m3 + 0.5 * m4
    rib_loss = 2.0 * _COMMIT * m2 + _COMMIT * forward
    p0 = hist_ref[0:1, :] * (1.0 / _B)
    p1 = hist_ref[1:2, :] * (1.0 / _B)
    e0 = -jnp.sum(p0 * jnp.log(p0 + 1e-10))
    e1 = -jnp.sum(p1 * jnp.log(p1 + 1e-10))
    out_ref[0] = scr_loss
    out_ref[1] = rib_loss
    out_ref[2] = jnp.exp(e0)
    out_ref[3] = jnp.exp(e1)


def _loss_pallas(scr, rib, qzs, qzr, hist):
    return pl.pallas_call(
        _loss_body,
        out_shape=jax.ShapeDtypeStruct((4,), jnp.float32),
        out_specs=pl.BlockSpec(memory_space=pltpu.SMEM),
    )(scr, rib, qzs, qzr, hist)


def kernel(scRNA_semantic, ribo_semantic, flag, embedding, proj_w, proj_b):
    quant_codebook = embedding @ proj_w.T + proj_b
    cb_sq = jnp.sum(quant_codebook ** 2, axis=1)
    x = jnp.concatenate([scRNA_semantic, ribo_semantic], axis=0)
    xsq = jnp.sum(x ** 2, axis=1, keepdims=True)

    idx = _argmin_pallas(-2.0 * x, xsq, cb_sq.reshape(1, _K), quant_codebook)
    idx2d = idx.reshape(_B2 // _CH, _CH)

    qz, hist = _sc_gather_hist(idx2d, quant_codebook)
    qz_s = qz[:_B]
    qz_r = qz[_B:]

    out = _loss_pallas(scRNA_semantic, ribo_semantic, qz_s, qz_r, hist)

    scRNA_loss = out[0]
    ribo_loss = out[1]
    scRNA_perplexity = out[2]
    ribo_perplexity = out[3]
    return (qz_s, qz_r, scRNA_loss, ribo_loss,
            scRNA_perplexity, ribo_perplexity)


# speedup vs baseline: 1.3289x; 1.0023x over previous
import functools

import jax
import jax.numpy as jnp
from jax import lax
from jax.experimental import pallas as pl
from jax.experimental.pallas import tpu as pltpu
from jax.experimental.pallas import tpu_sc as plsc

_B = 4096
_D = 64
_K = 8192
_B2 = 2 * _B
_COMMIT = 0.25

_BB = 8192
_KB = 512

_NC = 2
_NS = 16
_NW = _NC * _NS
_BPW = _B2 // _NW
_CH = 128
_NCH = _BPW // _CH
_ZW = _K // _NS


_W = 128


def _sweep_body(x2_ref, xsq_ref, cbsq_ref, cb_ref, bv_ref, bg_ref):
    k = pl.program_id(1)

    mm2 = lax.dot_general(
        x2_ref[...], cb_ref[...], (((1,), (1,)), ((), ())),
        preferred_element_type=jnp.float32)
    dist = (xsq_ref[...] + cbsq_ref[...]) + mm2
    d0 = dist[:, 0 * _W:1 * _W]
    d1 = dist[:, 1 * _W:2 * _W]
    d2 = dist[:, 2 * _W:3 * _W]
    d3 = dist[:, 3 * _W:4 * _W]
    m01 = jnp.minimum(d0, d1)
    i01 = jnp.where(d1 < d0, 1 * _W, 0 * _W)
    m23 = jnp.minimum(d2, d3)
    i23 = jnp.where(d3 < d2, 3 * _W, 2 * _W)
    m = jnp.minimum(m01, m23)
    i = jnp.where(m23 < m01, i23, i01)
    lane = lax.broadcasted_iota(jnp.int32, (_BB, _W), 1)
    g = (i + k * _KB) + lane
    better = (m < bv_ref[...]) | (k == 0)
    bv_ref[...] = jnp.where(better, m, bv_ref[...])
    bg_ref[...] = jnp.where(better, g, bg_ref[...])


def _sweep_pallas(x2, xsq, cbsq2d, qcb):
    return pl.pallas_call(
        _sweep_body,
        grid=(_B2 // _BB, _K // _KB),
        in_specs=[
            pl.BlockSpec((_BB, _D), lambda b, k: (b, 0)),
            pl.BlockSpec((_BB, 1), lambda b, k: (b, 0)),
            pl.BlockSpec((1, _KB), lambda b, k: (0, k)),
            pl.BlockSpec((_KB, _D), lambda b, k: (k, 0)),
        ],
        out_specs=[pl.BlockSpec((_BB, _W), lambda b, k: (b, 0)),
                   pl.BlockSpec((_BB, _W), lambda b, k: (b, 0))],
        out_shape=[jax.ShapeDtypeStruct((_B2, _W), jnp.float32),
                   jax.ShapeDtypeStruct((_B2, _W), jnp.int32)],
    )(x2, xsq, cbsq2d, qcb)


def _finish_body(bv_ref, bg_ref, idx_ref):
    bv = bv_ref[...]
    rowmin = jnp.min(bv, axis=1, keepdims=True)
    cand = jnp.where(bv == rowmin, bg_ref[...], jnp.int32(2**31 - 1))
    idx_ref[...] = jnp.min(cand, axis=1, keepdims=True)


def _finish_pallas(bv, bg):
    return pl.pallas_call(
        _finish_body,
        grid=(_B2 // _BB,),
        in_specs=[
            pl.BlockSpec((_BB, _W), lambda b: (b, 0)),
            pl.BlockSpec((_BB, _W), lambda b: (b, 0)),
        ],
        out_specs=pl.BlockSpec((_BB, 1), lambda b: (b, 0)),
        out_shape=jax.ShapeDtypeStruct((_B2, 1), jnp.int32),
    )(bv, bg)


def _argmin_pallas(x2, xsq, cbsq2d, qcb):
    bv, bk = _sweep_pallas(x2, xsq, cbsq2d, qcb)
    return _finish_pallas(bv, bk)


def _sc_body(idx_hbm, cb_hbm, out_hbm, hist_hbm,
             idx_v, rows_v, ones_v, zeros_v, hist_sh, sem):
    c = lax.axis_index("c")
    s = lax.axis_index("s")
    wid = c * _NS + s

    pltpu.sync_copy(idx_hbm.at[pl.ds(wid * _NCH, _NCH)], idx_v)

    cps = [
        pltpu.async_copy(cb_hbm.at[idx_v.at[j]],
                         rows_v.at[pl.ds(j * _CH, _CH)], sem)
        for j in range(_NCH)
    ]

    for i in range(_CH // 16):
        ones_v[pl.ds(i * 16, 16)] = jnp.ones((16,), jnp.float32)
    for i in range(_ZW // 16):
        zeros_v[pl.ds(i * 16, 16)] = jnp.zeros((16,), jnp.float32)
    pltpu.sync_copy(zeros_v, hist_sh.at[pl.ds(s * _ZW, _ZW)])
    plsc.subcore_barrier()
    for j in range(_NCH):
        pltpu.sync_copy(ones_v, hist_sh.at[idx_v.at[j]], add=True)
    plsc.subcore_barrier()

    for cp in cps:
        cp.wait()
    pltpu.sync_copy(rows_v, out_hbm.at[pl.ds(wid * _BPW, _BPW)])

    @pl.when(s == 0)
    def _write_hist():
        pltpu.sync_copy(hist_sh, hist_hbm.at[c])


def _sc_gather_hist(idx2d, qcb):
    mesh = plsc.VectorSubcoreMesh(core_axis_name="c", subcore_axis_name="s")
    run = pl.kernel(
        _sc_body,
        out_type=(
            jax.ShapeDtypeStruct((_B2, _D), jnp.float32),
            jax.ShapeDtypeStruct((_NC, _K), jnp.float32),
        ),
        mesh=mesh,
        scratch_types=[
            pltpu.VMEM((_NCH, _CH), jnp.int32),
            pltpu.VMEM((_BPW, _D), jnp.float32),
            pltpu.VMEM((_CH,), jnp.float32),
            pltpu.VMEM((_ZW,), jnp.float32),
            pltpu.VMEM_SHARED((_K,), jnp.float32),
            pltpu.SemaphoreType.DMA,
        ],
        compiler_params=pltpu.CompilerParams(use_tc_tiling_on_sc=False),
    )
    return run(idx2d, qcb)


def _loss_body(scr_ref, rib_ref, qzs_ref, qzr_ref, hist_ref, out_ref):
    scr = scr_ref[...]
    rib = rib_ref[...]
    qzs = qzs_ref[...]
    qzr = qzr_ref[...]
    n = float(_B * _D)
    m1 = jnp.sum((scr - qzs) ** 2) / n
    m2 = jnp.sum((rib - qzr) ** 2) / n
    m3 = jnp.sum((qzr - scr) ** 2) / n
    m4 = jnp.sum((qzs - rib) ** 2) / n
    scr_loss = 2.0 * _COMMIT * m1
    forward = m2 + m1 + 0.5 * m3 + 0.5 * m4
    rib_loss = 2.0 * _COMMIT * m2 + _COMMIT * forward
    p0 = hist_ref[0:1, :] * (1.0 / _B)
    p1 = hist_ref[1:2, :] * (1.0 / _B)
    e0 = -jnp.sum(p0 * jnp.log(p0 + 1e-10))
    e1 = -jnp.sum(p1 * jnp.log(p1 + 1e-10))
    out_ref[0] = scr_loss
    out_ref[1] = rib_loss
    out_ref[2] = jnp.exp(e0)
    out_ref[3] = jnp.exp(e1)


def _loss_pallas(scr, rib, qzs, qzr, hist):
    return pl.pallas_call(
        _loss_body,
        out_shape=jax.ShapeDtypeStruct((4,), jnp.float32),
        out_specs=pl.BlockSpec(memory_space=pltpu.SMEM),
    )(scr, rib, qzs, qzr, hist)


def kernel(scRNA_semantic, ribo_semantic, flag, embedding, proj_w, proj_b):
    quant_codebook = embedding @ proj_w.T + proj_b
    cb_sq = jnp.sum(quant_codebook ** 2, axis=1)
    x = jnp.concatenate([scRNA_semantic, ribo_semantic], axis=0)
    xsq = jnp.sum(x ** 2, axis=1, keepdims=True)

    idx = _argmin_pallas(-2.0 * x, xsq, cb_sq.reshape(1, _K), quant_codebook)
    idx2d = idx.reshape(_B2 // _CH, _CH)

    qz, hist = _sc_gather_hist(idx2d, quant_codebook)
    qz_s = qz[:_B]
    qz_r = qz[_B:]

    out = _loss_pallas(scRNA_semantic, ribo_semantic, qz_s, qz_r, hist)

    scRNA_loss = out[0]
    ribo_loss = out[1]
    scRNA_perplexity = out[2]
    ribo_perplexity = out[3]
    return (qz_s, qz_r, scRNA_loss, ribo_loss,
            scRNA_perplexity, ribo_perplexity)
